# baseline (speedup 1.0000x reference)
import jax
import jax.numpy as jnp
from jax.experimental import pallas as pl

_N_NODES = 50000


def _nrm(x, axis=-1, keepdims=False, eps=1e-8, sqrt=True):
    out = jnp.maximum(jnp.sum(jnp.square(x), axis=axis, keepdims=keepdims), eps)
    return jnp.sqrt(out) if sqrt else out


def _klin(p, x):
    y = x @ p["w"].T
    if "b" in p:
        y = y + p["b"]
    return y


def _kgvp(p, x, in_dims, out_dims, acts=("relu", "sigmoid")):
    si, vi = in_dims
    so, vo = out_dims
    sact, vact = acts
    vout = None
    if vi:
        s, v = x
        vt = jnp.swapaxes(v, -1, -2)
        vh = _klin(p["wh"], vt)
        vn = _nrm(vh, axis=-2)
        s = _klin(p["ws"], jnp.concatenate([s, vn], axis=-1))
        if vo:
            vout = jnp.swapaxes(_klin(p["wv"], vh), -1, -2)
            if vact == "sigmoid":
                vout = vout * jax.nn.sigmoid(_nrm(vout, axis=-1, keepdims=True))
    else:
        s = _klin(p["ws"], x)
        if vo:
            vout = jnp.zeros(s.shape[:-1] + (vo, 3), s.dtype)
    if sact == "relu":
        s = jax.nn.relu(s)
    return (s, vout) if vo else s


def _kln_s(p, s):
    mu = jnp.mean(s, axis=-1, keepdims=True)
    var = jnp.mean(jnp.square(s - mu), axis=-1, keepdims=True)
    return (s - mu) / jnp.sqrt(var + 1e-5) * p["g"] + p["b"]


def _kln_sv(p, s, v):
    vn = _nrm(v, axis=-1, keepdims=True, sqrt=False)
    vn = jnp.sqrt(jnp.mean(vn, axis=-2, keepdims=True))
    return _kln_s(p, s), v / vn


def _kconv(p, s, v, edge_index, e_s, e_v):
    src, dst = edge_index[0], edge_index[1]
    ms = jnp.concatenate([s[src], e_s, s[dst]], axis=-1)
    mv = jnp.concatenate([v[src], e_v, v[dst]], axis=1)
    m = _kgvp(p["m0"], (ms, mv), (232, 33), (100, 16))
    m = _kgvp(p["m1"], m, (100, 16), (100, 16))
    m_s, m_v = _kgvp(p["m2"], m, (100, 16), (100, 16), acts=(None, None))
    sum_s = jax.ops.segment_sum(m_s, dst, num_segments=_N_NODES)
    sum_v = jax.ops.segment_sum(m_v.reshape(m_v.shape[0], -1), dst, num_segments=_N_NODES).reshape(_N_NODES, 16, 3)
    cnt = jnp.maximum(jax.ops.segment_sum(jnp.ones((m_s.shape[0],), m_s.dtype), dst, num_segments=_N_NODES), 1.0)
    return sum_s / cnt[:, None], sum_v / cnt[:, None, None]


def kernel(params, h_V_s, h_V_v, edge_index, h_E_s, h_E_v, seq):
    seq_emb = jnp.take(params["W_s"], seq, axis=0)
    s = jnp.concatenate([h_V_s, seq_emb], axis=-1)
    v = h_V_v
    s, v = _kln_sv(params["Wv_ln"], s, v)
    s, v = _kgvp(params["Wv_gvp"], (s, v), (26, 3), (100, 16), acts=(None, None))
    e_s, e_v = _kln_sv(params["We_ln"], h_E_s, h_E_v)
    e_s, e_v = _kgvp(params["We_gvp"], (e_s, e_v), (32, 1), (32, 1), acts=(None, None))
    for i in range(3):
        lp = params["layer%d" % i]
        ds, dv = _kconv(lp["conv"], s, v, edge_index, e_s, e_v)
        s, v = _kln_sv(lp["norm0"], s + ds, v + dv)
        fs, fv = _kgvp(lp["ff0"], (s, v), (100, 16), (400, 32))
        fs, fv = _kgvp(lp["ff1"], (fs, fv), (400, 32), (100, 16), acts=(None, None))
        s, v = _kln_sv(lp["norm1"], s + fs, v + fv)
    s, v = _kln_sv(params["Wout_ln"], s, v)
    out = _kgvp(params["Wout_gvp"], (s, v), (100, 16), (100, 0))
    return out


# speedup vs baseline: 1.4452x; 1.4452x over previous
import functools

import jax
import jax.numpy as jnp
from jax import lax
from jax.experimental import pallas as pl
from jax.experimental.pallas import tpu as pltpu

try:
    from jax.experimental.pallas import tpu_sc as plsc
    _HAS_SC = True
except ImportError:
    _HAS_SC = False

N_NODES_K = 50000
N_EDGES_K = 800000
NPAD = 50176
EPAD = 819200
RNG = 13056
ACCR = RNG + 128
NACC = 4 * RNG
TW = 128
SW = 64
NSTRIP = 3
EPW = EPAD // 32
CCH = 128
BN = 512
BE = 512

_EPS = 1e-8


def _sq(x):
    return x * x


def _vnorm3(vf, nc, eps=_EPS):
    s = _sq(vf[:, :nc]) + _sq(vf[:, nc:2 * nc]) + _sq(vf[:, 2 * nc:3 * nc])
    return jnp.sqrt(jnp.maximum(s, eps))


def _tile3(g, nc):
    return jnp.concatenate([g, g, g], axis=-1)


def _ln_scalar(s, g, b, nvalid):
    mu = jnp.mean(s, axis=-1, keepdims=True)
    var = jnp.mean(_sq(s - mu), axis=-1, keepdims=True)
    return (s - mu) / jnp.sqrt(var + 1e-5) * g + b


def _ln_vec(vf, nc):
    n2 = jnp.maximum(_sq(vf[:, :nc]) + _sq(vf[:, nc:2 * nc]) + _sq(vf[:, 2 * nc:]), _EPS)
    vn = jnp.sqrt(jnp.mean(n2, axis=-1, keepdims=True))
    return vf / vn


def _kron3(w):
    return jnp.kron(jnp.eye(3, dtype=w.dtype), w.T)


def _pack_pairs(v):
    u = lax.bitcast_convert_type(v, jnp.uint32)
    hi = (u[:, :24] + 0x8000) & jnp.uint32(0xFFFF0000)
    lo = (u[:, 24:] + 0x8000) >> 16
    return lax.bitcast_convert_type(hi | lo, jnp.float32)


def _unpack_pairs(p):
    u = lax.bitcast_convert_type(p, jnp.uint32)
    hi = lax.bitcast_convert_type(u & jnp.uint32(0xFFFF0000), jnp.float32)
    lo = lax.bitcast_convert_type(u << 16, jnp.float32)
    return jnp.concatenate([hi, lo], axis=-1)


def _node0_body(hs_ref, soh_ref, vf_ref, wsemb_ref, g_ref, b_ref,
                wh_ref, wsw_ref, wsb_ref, wv_ref, wsrc_ref, wdst_ref,
                s_ref, v_ref, ts_ref, td_ref):
    seq_emb = soh_ref[...] @ wsemb_ref[...]
    s = jnp.concatenate([hs_ref[:, :6], seq_emb], axis=-1)
    s = _ln_scalar(s, g_ref[0, :26], b_ref[0, :26], 26)
    vf = _ln_vec(vf_ref[:, :9], 3)
    vh = vf @ wh_ref[...]
    vn = _vnorm3(vh, 16)
    so = jnp.concatenate([s, vn], axis=-1) @ wsw_ref[...] + wsb_ref[0]
    vo = vh @ wv_ref[...]
    s_ref[...] = so
    v_ref[...] = vo
    vp = _pack_pairs(vo)
    pad = jnp.zeros((so.shape[0], TW - 124), so.dtype)
    ts_ref[...] = jnp.concatenate([so @ wsrc_ref[...], vp, pad], axis=-1)
    td_ref[...] = jnp.concatenate([so @ wdst_ref[...], vp, pad], axis=-1)


def _edge0_body(es_ref, evf_ref, g_ref, b_ref, wh_ref, wsw_ref, wsb_ref,
                wv_ref, ef_ref):
    s = _ln_scalar(es_ref[...], g_ref[0], b_ref[0], 32)
    vf = _ln_vec(evf_ref[:, :3], 1)
    vh = vf * wh_ref[0, 0]
    vn = jnp.sqrt(jnp.maximum(_sq(vh[:, :1]) + _sq(vh[:, 1:2]) + _sq(vh[:, 2:3]), _EPS))
    so = jnp.concatenate([s, vn], axis=-1) @ wsw_ref[...] + wsb_ref[0]
    vo = vh * wv_ref[0, 0]
    pad = jnp.zeros((so.shape[0], 64 - 35), so.dtype)
    ef_ref[...] = jnp.concatenate([so, vo, pad], axis=-1)


def _msg_body(gs_ref, gd_ref, ef_ref,
              k0s_ref, k0e_ref, k0d_ref, wes_ref, wd0_ref, b0_ref, kv0_ref,
              k1h_ref, w1s_ref, w1n_ref, b1_ref, kv1_ref,
              k2h_ref, w2s_ref, w2n_ref, b2_ref, kv2_ref,
              *o_refs):
    gs = gs_ref[...]
    gd = gd_ref[...]
    ef = ef_ref[...]
    vs = _unpack_pairs(gs[:, 100:124])
    vd = _unpack_pairs(gd[:, 100:124])
    ev = ef[:, 32:35]
    es = ef[:, :32]
    vh = vs @ k0s_ref[...] + ev @ k0e_ref[...] + vd @ k0d_ref[...]
    vn0 = _vnorm3(vh, 33)
    s1 = gs[:, :100] + gd[:, :100] + es @ wes_ref[...] + vn0 @ wd0_ref[...] + b0_ref[0]
    s1 = jnp.maximum(s1, 0.0)
    vo = vh @ kv0_ref[...]
    gate = jax.nn.sigmoid(_vnorm3(vo, 16))
    vo = vo * _tile3(gate, 16)
    vh1 = vo @ k1h_ref[...]
    vn1 = _vnorm3(vh1, 16)
    s2 = s1 @ w1s_ref[...] + vn1 @ w1n_ref[...] + b1_ref[0]
    s2 = jnp.maximum(s2, 0.0)
    vo1 = vh1 @ kv1_ref[...]
    gate1 = jax.nn.sigmoid(_vnorm3(vo1, 16))
    vo1 = vo1 * _tile3(gate1, 16)
    vh2 = vo1 @ k2h_ref[...]
    vn2 = _vnorm3(vh2, 16)
    s3 = s2 @ w2s_ref[...] + vn2 @ w2n_ref[...] + b2_ref[0]
    vo2 = vh2 @ kv2_ref[...]
    one = jnp.ones((s3.shape[0], 1), s3.dtype)
    pad = jnp.zeros((s3.shape[0], NSTRIP * SW - 149), s3.dtype)
    m = jnp.concatenate([s3, vo2, one, pad], axis=-1)
    for t, oref in enumerate(o_refs):
        oref[...] = m[:, t * SW:(t + 1) * SW]


def _upd_body(last, *refs):
    (sp_ref, vp_ref), strip_refs, rest = refs[:2], refs[2:2 + NSTRIP], refs[2 + NSTRIP:]
    (g0_ref, bb0_ref, kf0_ref, wf0_ref, bf0_ref, kvf0_ref,
     kf1_ref, wf1_ref, bf1_ref, kvf1_ref, g1_ref, bb1_ref,
     wsrc_ref, wdst_ref) = rest[:14]
    outs = rest[14:]
    if last:
        s_ref, v_ref = outs
        ts_ref = td_ref = None
    else:
        s_ref, v_ref, ts_ref, td_ref = outs
    msum = jnp.concatenate([r[...] for r in strip_refs], axis=-1)
    cnt = jnp.maximum(msum[:, 148:149], 1.0)
    s = sp_ref[...] + msum[:, :100] / cnt
    vf = vp_ref[...] + msum[:, 100:148] / cnt
    s = _ln_scalar(s, g0_ref[0], bb0_ref[0], 100)
    vf = _ln_vec(vf, 16)
    vh = vf @ kf0_ref[...]
    vn = _vnorm3(vh, 32)
    fs = jnp.maximum(jnp.concatenate([s, vn], axis=-1) @ wf0_ref[...] + bf0_ref[0], 0.0)
    fv = vh @ kvf0_ref[...]
    gate = jax.nn.sigmoid(_vnorm3(fv, 32))
    fv = fv * _tile3(gate, 32)
    vh1 = fv @ kf1_ref[...]
    vn1 = _vnorm3(vh1, 32)
    fs1 = jnp.concatenate([fs, vn1], axis=-1) @ wf1_ref[...] + bf1_ref[0]
    fv1 = vh1 @ kvf1_ref[...]
    s = s + fs1
    vf = vf + fv1
    s = _ln_scalar(s, g1_ref[0], bb1_ref[0], 100)
    vf = _ln_vec(vf, 16)
    s_ref[...] = s
    v_ref[...] = vf
    if not last:
        vp = _pack_pairs(vf)
        pad = jnp.zeros((s.shape[0], TW - 124), s.dtype)
        ts_ref[...] = jnp.concatenate([s @ wsrc_ref[...], vp, pad], axis=-1)
        td_ref[...] = jnp.concatenate([s @ wdst_ref[...], vp, pad], axis=-1)


def _out_body(s_ref, v_ref, g_ref, b_ref, wh_ref, wsw_ref, wsb_ref, o_ref):
    s = _ln_scalar(s_ref[...], g_ref[0], b_ref[0], 100)
    vf = _ln_vec(v_ref[...], 16)
    vh = vf @ wh_ref[...]
    vn = _vnorm3(vh, 16)
    so = jnp.concatenate([s, vn], axis=-1) @ wsw_ref[...] + wsb_ref[0]
    o_ref[...] = jnp.maximum(so, 0.0)


def _full(shape):
    return pl.BlockSpec(shape, lambda i: (0,) * len(shape))


def _rows(w, blk=None):
    return pl.BlockSpec((blk or BN, w), lambda i: (i, 0))


def _tc(body, grid, in_specs, out_specs, out_shape):
    return pl.pallas_call(
        body, grid=(grid,), in_specs=in_specs, out_specs=out_specs,
        out_shape=out_shape,
        compiler_params=pltpu.CompilerParams(
            dimension_semantics=("arbitrary",)),
    )


def _sc_gather(tsrc, tdst, srcp, dstp):
    mesh = plsc.VectorSubcoreMesh(core_axis_name="c", subcore_axis_name="s")

    @functools.partial(
        pl.kernel, mesh=mesh,
        out_type=[jax.ShapeDtypeStruct((EPAD, TW), jnp.float32),
                  jax.ShapeDtypeStruct((EPAD, TW), jnp.float32)],
        scratch_types=[pltpu.VMEM((CCH,), jnp.int32),
                       pltpu.VMEM((CCH,), jnp.int32),
                       pltpu.VMEM((CCH, TW), jnp.float32),
                       pltpu.VMEM((CCH, TW), jnp.float32),
                       pltpu.SemaphoreType.DMA,
                       pltpu.SemaphoreType.DMA],
    )
    def k(ts_hbm, td_hbm, src_hbm, dst_hbm, os_hbm, od_hbm,
          idx_s, idx_d, row_s, row_d, sem_s, sem_d):
        wid = lax.axis_index("s") * 2 + lax.axis_index("c")
        base = wid * EPW

        def body(j, carry):
            off = base + j * CCH
            pltpu.sync_copy(src_hbm.at[pl.ds(off, CCH)], idx_s)
            pltpu.sync_copy(dst_hbm.at[pl.ds(off, CCH)], idx_d)
            cp_s = pltpu.async_copy(ts_hbm.at[idx_s], row_s, sem_s)
            cp_d = pltpu.async_copy(td_hbm.at[idx_d], row_d, sem_d)
            cp_s.wait()
            cp_d.wait()
            pltpu.sync_copy(row_s, os_hbm.at[pl.ds(off, CCH)])
            pltpu.sync_copy(row_d, od_hbm.at[pl.ds(off, CCH)])
            return carry

        lax.fori_loop(0, EPW // CCH, body, 0)

    return k(tsrc, tdst, srcp, dstp)


_NBUF = 2
_ZCH = 104
_FCH = 136


def _sc_scatter(dstp, strips, zeros_hbm):
    mesh = plsc.VectorSubcoreMesh(core_axis_name="c", subcore_axis_name="s")
    chunks_per_tile = EPAD // 16 // CCH

    @functools.partial(
        pl.kernel, mesh=mesh,
        out_type=[jax.ShapeDtypeStruct((NACC, SW), jnp.float32)
                  for _ in range(NSTRIP)],
        scratch_types=(
            [pltpu.VMEM((CCH,), jnp.int32) for _ in range(_NBUF)]
            + [pltpu.VMEM((CCH, SW), jnp.float32) for _ in range(_NBUF)]
            + [pltpu.VMEM((_ZCH, SW), jnp.float32),
               pltpu.VMEM((_FCH, SW), jnp.float32),
               pltpu.VMEM_SHARED((ACCR, SW), jnp.float32)]
            + [pltpu.SemaphoreType.DMA for _ in range(2 * _NBUF + 1)]),
    )
    def k(*refs):
        dst_hbm = refs[0]
        ins = refs[1:1 + NSTRIP]
        z_hbm = refs[1 + NSTRIP]
        outs = refs[2 + NSTRIP:2 + 2 * NSTRIP]
        scr = refs[2 + 2 * NSTRIP:]
        idxs = scr[:_NBUF]
        msgs = scr[_NBUF:2 * _NBUF]
        zbuf, fbuf, acc = scr[2 * _NBUF:2 * _NBUF + 3]
        isems = scr[2 * _NBUF + 3:3 * _NBUF + 3]
        msems = scr[3 * _NBUF + 3:4 * _NBUF + 3]
        ssem = scr[4 * _NBUF + 3]
        cid = lax.axis_index("c")
        sid = lax.axis_index("s")
        pltpu.sync_copy(z_hbm, zbuf)
        for st in range(NSTRIP):
            for rr in range(2):
                base = (cid * 2 + rr) * RNG
                tb = sid * (ACCR // 16)
                for h in range(7):
                    pltpu.sync_copy(zbuf, acc.at[pl.ds(tb + h * _ZCH, _ZCH)])
                pltpu.sync_copy(zbuf.at[pl.ds(0, 96)],
                                acc.at[pl.ds(tb + 7 * _ZCH, 96)])
                plsc.subcore_barrier()

                def body(jj, carry, st=st, base=base):
                    cbase = sid * (EPAD // 16) + jj * (_NBUF * CCH)
                    loads = []
                    for b in range(_NBUF):
                        off = cbase + b * CCH
                        li = pltpu.async_copy(dst_hbm.at[pl.ds(off, CCH)],
                                              idxs[b], isems[b])
                        lm = pltpu.async_copy(ins[st].at[pl.ds(off, CCH)],
                                              msgs[b], msems[b])
                        loads.append((li, lm))
                    adds = []
                    for b in range(_NBUF):
                        li, lm = loads[b]
                        li.wait()
                        for q in range(CCH // 16):
                            v = idxs[b][pl.ds(q * 16, 16)]
                            rel = v - base
                            oob = (rel < 0) | (rel >= RNG)
                            idxs[b][pl.ds(q * 16, 16)] = jnp.where(
                                oob, RNG + (v & 127), rel)
                        lm.wait()
                        adds.append(pltpu.async_copy(msgs[b], acc.at[idxs[b]],
                                                     ssem, add=True))
                    for b in range(_NBUF):
                        adds[b].wait()
                    return carry

                lax.fori_loop(0, chunks_per_tile // _NBUF, body, 0)
                plsc.subcore_barrier()
                for h in range(6):
                    arow = sid * (6 * _FCH) + h * _FCH
                    pltpu.sync_copy(acc.at[pl.ds(arow, _FCH)], fbuf)
                    pltpu.sync_copy(fbuf, outs[st].at[pl.ds(base + arow, _FCH)])
                plsc.subcore_barrier()

    return k(dstp, *strips, zeros_hbm)


def _sc_scatter_bisect(dstp, strips, zeros_hbm):
    del zeros_hbm
    return [jax.ops.segment_sum(st, dstp, num_segments=NACC) for st in strips]


def _pad_rows(x, n):
    return jnp.pad(x, ((0, n - x.shape[0]),) + ((0, 0),) * (x.ndim - 1))


def kernel(params, h_V_s, h_V_v, edge_index, h_E_s, h_E_v, seq):
    f32 = jnp.float32
    p = params
    ngrid = NPAD // BN
    egrid = EPAD // BE

    hs = _pad_rows(h_V_s, NPAD)
    hs = jnp.pad(hs, ((0, 0), (0, 2)))
    vf0 = _pad_rows(h_V_v.transpose(0, 2, 1).reshape(-1, 9), NPAD)
    vf0 = jnp.pad(vf0, ((0, 0), (0, 7)))
    seqp = _pad_rows(seq[:, None], NPAD)[:, 0]
    soh = (seqp[:, None] == jnp.arange(24, dtype=jnp.int32)[None, :]).astype(f32)
    es_in = _pad_rows(h_E_s, EPAD)
    evf = _pad_rows(h_E_v[:, 0, :], EPAD)
    evf = jnp.pad(evf, ((0, 0), (0, 5)))
    srcp = jnp.pad(edge_index[0], (0, EPAD - N_EDGES_K))
    npad_e = EPAD - N_EDGES_K
    pad_rows = N_NODES_K + jnp.arange(npad_e, dtype=jnp.int32) % (NACC - N_NODES_K)
    dstp = jnp.concatenate([edge_index[1], pad_rows])
    zeros_hbm = jnp.zeros((_ZCH, SW), f32)

    wsemb = jnp.pad(p["W_s"], ((0, 4), (0, 0)))

    gvp0 = p["Wv_gvp"]
    node0_w = (wsemb,
               p["Wv_ln"]["g"][None, :].astype(f32), p["Wv_ln"]["b"][None, :],
               _kron3(gvp0["wh"]["w"]),
               gvp0["ws"]["w"].T, gvp0["ws"]["b"][None, :],
               _kron3(gvp0["wv"]["w"]))

    gvpe = p["We_gvp"]
    edge0_w = (p["We_ln"]["g"][None, :], p["We_ln"]["b"][None, :],
               gvpe["wh"]["w"], gvpe["ws"]["w"].T, gvpe["ws"]["b"][None, :],
               gvpe["wv"]["w"])

    def layer_w(i):
        lp = p["layer%d" % i]["conv"]
        m0, m1, m2 = lp["m0"], lp["m1"], lp["m2"]
        wh0 = m0["wh"]["w"]
        ws0 = m0["ws"]["w"]
        return dict(
            wsrc=ws0[:, 0:100].T, wdst=ws0[:, 132:232].T,
            wes=ws0[:, 100:132].T,
            wd0=ws0[:, 232:265].T,
            k0s=_kron3(wh0[:, 0:16]),
            k0e=jnp.kron(jnp.eye(3, dtype=f32), wh0[:, 16:17].T),
            k0d=_kron3(wh0[:, 17:33]),
            b0v=m0["ws"]["b"][None, :],
            kv0=_kron3(m0["wv"]["w"]),
            k1h=_kron3(m1["wh"]["w"]),
            w1s=m1["ws"]["w"][:, 0:100].T, w1n=m1["ws"]["w"][:, 100:116].T,
            b1=m1["ws"]["b"][None, :],
            kv1=_kron3(m1["wv"]["w"]),
            k2h=_kron3(m2["wh"]["w"]),
            w2s=m2["ws"]["w"][:, 0:100].T, w2n=m2["ws"]["w"][:, 100:116].T,
            b2=m2["ws"]["b"][None, :],
            kv2=_kron3(m2["wv"]["w"]),
        )

    def ff_w(i, nxt):
        lp = p["layer%d" % i]
        f0, f1 = lp["ff0"], lp["ff1"]
        w = dict(
            g0=lp["norm0"]["g"][None, :], bb0=lp["norm0"]["b"][None, :],
            kf0=_kron3(f0["wh"]["w"]),
            wf0=f0["ws"]["w"].T, bf0=f0["ws"]["b"][None, :],
            kvf0=_kron3(f0["wv"]["w"]),
            kf1=_kron3(f1["wh"]["w"]),
            wf1=f1["ws"]["w"].T, bf1=f1["ws"]["b"][None, :],
            kvf1=_kron3(f1["wv"]["w"]),
            g1=lp["norm1"]["g"][None, :], bb1=lp["norm1"]["b"][None, :],
        )
        if nxt is not None:
            ws0n = p["layer%d" % nxt]["conv"]["m0"]["ws"]["w"]
            w["wsrc"] = ws0n[:, 0:100].T
            w["wdst"] = ws0n[:, 132:232].T
        else:
            w["wsrc"] = jnp.zeros((100, 100), f32)
            w["wdst"] = jnp.zeros((100, 100), f32)
        return w

    lw0 = layer_w(0)
    s, v, tsrc, tdst = _tc(
        _node0_body, ngrid,
        [_rows(8), _rows(24), _rows(16)] + [_full(w.shape) for w in (
            node0_w[0], node0_w[1], node0_w[2], node0_w[3], node0_w[4],
            node0_w[5], node0_w[6], lw0["wsrc"], lw0["wdst"])],
        [_rows(100), _rows(48), _rows(TW), _rows(TW)],
        [jax.ShapeDtypeStruct((NPAD, 100), f32),
         jax.ShapeDtypeStruct((NPAD, 48), f32),
         jax.ShapeDtypeStruct((NPAD, TW), f32),
         jax.ShapeDtypeStruct((NPAD, TW), f32)],
    )(hs, soh, vf0, *node0_w, lw0["wsrc"], lw0["wdst"])

    ef = _tc(
        _edge0_body, egrid,
        [_rows(32, BE), _rows(8, BE)] + [_full(w.shape) for w in edge0_w],
        [_rows(64, BE)],
        [jax.ShapeDtypeStruct((EPAD, 64), f32)],
    )(es_in, evf, *edge0_w)[0]

    for i in range(3):
        lw = layer_w(i)
        gs, gd = _sc_gather(tsrc, tdst, srcp, dstp)
        mw = [lw[k] for k in ("k0s", "k0e", "k0d", "wes", "wd0", "b0v", "kv0",
                              "k1h", "w1s", "w1n", "b1", "kv1",
                              "k2h", "w2s", "w2n", "b2", "kv2")]
        strips = _tc(
            _msg_body, egrid,
            [_rows(TW, BE), _rows(TW, BE), _rows(64, BE)] +
            [_full(w.shape) for w in mw],
            [_rows(SW, BE)] * NSTRIP,
            [jax.ShapeDtypeStruct((EPAD, SW), f32)] * NSTRIP,
        )(gs, gd, ef, *mw)
        acc = _sc_scatter_bisect(dstp, strips, zeros_hbm)
        accs = [a[:NPAD] for a in acc]
        fw = ff_w(i, i + 1 if i < 2 else None)
        uw = [fw[k] for k in ("g0", "bb0", "kf0", "wf0", "bf0", "kvf0",
                              "kf1", "wf1", "bf1", "kvf1", "g1", "bb1",
                              "wsrc", "wdst")]
        last = i == 2
        outs = _tc(
            functools.partial(_upd_body, last), ngrid,
            [_rows(100), _rows(48)] + [_rows(SW)] * NSTRIP +
            [_full(w.shape) for w in uw],
            [_rows(100), _rows(48)] + ([] if last else [_rows(TW), _rows(TW)]),
            [jax.ShapeDtypeStruct((NPAD, 100), f32),
             jax.ShapeDtypeStruct((NPAD, 48), f32)] +
            ([] if last else [jax.ShapeDtypeStruct((NPAD, TW), f32),
                              jax.ShapeDtypeStruct((NPAD, TW), f32)]),
        )(s, v, *accs, *uw)
        if last:
            s, v = outs
        else:
            s, v, tsrc, tdst = outs

    og = p["Wout_gvp"]
    ow = (p["Wout_ln"]["g"][None, :], p["Wout_ln"]["b"][None, :],
          _kron3(og["wh"]["w"]), og["ws"]["w"].T, og["ws"]["b"][None, :])
    out = _tc(
        _out_body, ngrid,
        [_rows(100), _rows(48)] + [_full(w.shape) for w in ow],
        [_rows(100)],
        [jax.ShapeDtypeStruct((NPAD, 100), f32)],
    )(s, v, *ow)[0]
    return out[:N_NODES_K]


# speedup vs baseline: 1.7821x; 1.2331x over previous
import functools

import jax
import jax.numpy as jnp
from jax import lax
from jax.experimental import pallas as pl
from jax.experimental.pallas import tpu as pltpu

try:
    from jax.experimental.pallas import tpu_sc as plsc
    _HAS_SC = True
except ImportError:
    _HAS_SC = False

N_NODES_K = 50000
N_EDGES_K = 800000
NPAD = 50176
EPAD = 819200
RNG = 13056
ACCR = RNG + 128
NACC = 4 * RNG
TW = 128
SW = 192
NSTRIP = 1
EPW = EPAD // 32
CCH = 128
BN = 512
BE = 512

_EPS = 1e-8


def _sq(x):
    return x * x


def _vnorm3(vf, nc, eps=_EPS):
    s = _sq(vf[:, :nc]) + _sq(vf[:, nc:2 * nc]) + _sq(vf[:, 2 * nc:3 * nc])
    return jnp.sqrt(jnp.maximum(s, eps))


def _tile3(g, nc):
    return jnp.concatenate([g, g, g], axis=-1)


def _ln_scalar(s, g, b, nvalid):
    mu = jnp.mean(s, axis=-1, keepdims=True)
    var = jnp.mean(_sq(s - mu), axis=-1, keepdims=True)
    return (s - mu) / jnp.sqrt(var + 1e-5) * g + b


def _ln_vec(vf, nc):
    n2 = jnp.maximum(_sq(vf[:, :nc]) + _sq(vf[:, nc:2 * nc]) + _sq(vf[:, 2 * nc:]), _EPS)
    vn = jnp.sqrt(jnp.mean(n2, axis=-1, keepdims=True))
    return vf / vn


def _kron3(w):
    return jnp.kron(jnp.eye(3, dtype=w.dtype), w.T)


def _pack_pairs(v):
    u = lax.bitcast_convert_type(v, jnp.uint32)
    hi = (u[:, :24] + 0x8000) & jnp.uint32(0xFFFF0000)
    lo = (u[:, 24:] + 0x8000) >> 16
    return lax.bitcast_convert_type(hi | lo, jnp.float32)


def _unpack_pairs(p):
    u = lax.bitcast_convert_type(p, jnp.uint32)
    hi = lax.bitcast_convert_type(u & jnp.uint32(0xFFFF0000), jnp.float32)
    lo = lax.bitcast_convert_type(u << 16, jnp.float32)
    return jnp.concatenate([hi, lo], axis=-1)


def _node0_body(hs_ref, soh_ref, vf_ref, wsemb_ref, g_ref, b_ref,
                wh_ref, wsw_ref, wsb_ref, wv_ref, wsrc_ref, wdst_ref,
                s_ref, v_ref, ts_ref, td_ref):
    seq_emb = soh_ref[...] @ wsemb_ref[...]
    s = jnp.concatenate([hs_ref[:, :6], seq_emb], axis=-1)
    s = _ln_scalar(s, g_ref[0, :26], b_ref[0, :26], 26)
    vf = _ln_vec(vf_ref[:, :9], 3)
    vh = vf @ wh_ref[...]
    vn = _vnorm3(vh, 16)
    so = jnp.concatenate([s, vn], axis=-1) @ wsw_ref[...] + wsb_ref[0]
    vo = vh @ wv_ref[...]
    s_ref[...] = so
    v_ref[...] = vo
    vp = _pack_pairs(vo)
    pad = jnp.zeros((so.shape[0], TW - 124), so.dtype)
    ts_ref[...] = jnp.concatenate([so @ wsrc_ref[...], vp, pad], axis=-1)
    td_ref[...] = jnp.concatenate([so @ wdst_ref[...], vp, pad], axis=-1)


def _edge0_body(es_ref, evf_ref, g_ref, b_ref, wh_ref, wsw_ref, wsb_ref,
                wv_ref, ef_ref):
    s = _ln_scalar(es_ref[...], g_ref[0], b_ref[0], 32)
    vf = _ln_vec(evf_ref[:, :3], 1)
    vh = vf * wh_ref[0, 0]
    vn = jnp.sqrt(jnp.maximum(_sq(vh[:, :1]) + _sq(vh[:, 1:2]) + _sq(vh[:, 2:3]), _EPS))
    so = jnp.concatenate([s, vn], axis=-1) @ wsw_ref[...] + wsb_ref[0]
    vo = vh * wv_ref[0, 0]
    pad = jnp.zeros((so.shape[0], 64 - 35), so.dtype)
    ef_ref[...] = jnp.concatenate([so, vo, pad], axis=-1)


def _msg_body(gs_ref, gd_ref, ef_ref,
              k0s_ref, k0e_ref, k0d_ref, wes_ref, wd0_ref, b0_ref, kv0_ref,
              k1h_ref, w1s_ref, w1n_ref, b1_ref, kv1_ref,
              k2h_ref, w2s_ref, w2n_ref, b2_ref, kv2_ref,
              *o_refs):
    gs = gs_ref[...]
    gd = gd_ref[...]
    ef = ef_ref[...]
    vs = _unpack_pairs(gs[:, 100:124])
    vd = _unpack_pairs(gd[:, 100:124])
    ev = ef[:, 32:35]
    es = ef[:, :32]
    vh = vs @ k0s_ref[...] + ev @ k0e_ref[...] + vd @ k0d_ref[...]
    vn0 = _vnorm3(vh, 33)
    s1 = gs[:, :100] + gd[:, :100] + es @ wes_ref[...] + vn0 @ wd0_ref[...] + b0_ref[0]
    s1 = jnp.maximum(s1, 0.0)
    vo = vh @ kv0_ref[...]
    gate = jax.nn.sigmoid(_vnorm3(vo, 16))
    vo = vo * _tile3(gate, 16)
    vh1 = vo @ k1h_ref[...]
    vn1 = _vnorm3(vh1, 16)
    s2 = s1 @ w1s_ref[...] + vn1 @ w1n_ref[...] + b1_ref[0]
    s2 = jnp.maximum(s2, 0.0)
    vo1 = vh1 @ kv1_ref[...]
    gate1 = jax.nn.sigmoid(_vnorm3(vo1, 16))
    vo1 = vo1 * _tile3(gate1, 16)
    vh2 = vo1 @ k2h_ref[...]
    vn2 = _vnorm3(vh2, 16)
    s3 = s2 @ w2s_ref[...] + vn2 @ w2n_ref[...] + b2_ref[0]
    vo2 = vh2 @ kv2_ref[...]
    one = jnp.ones((s3.shape[0], 1), s3.dtype)
    pad = jnp.zeros((s3.shape[0], NSTRIP * SW - 149), s3.dtype)
    m = jnp.concatenate([s3, vo2, one, pad], axis=-1)
    for t, oref in enumerate(o_refs):
        oref[...] = m[:, t * SW:(t + 1) * SW]


def _upd_body(last, *refs):
    (sp_ref, vp_ref), strip_refs, rest = refs[:2], refs[2:2 + NSTRIP], refs[2 + NSTRIP:]
    (g0_ref, bb0_ref, kf0_ref, wf0_ref, bf0_ref, kvf0_ref,
     kf1_ref, wf1_ref, bf1_ref, kvf1_ref, g1_ref, bb1_ref,
     wsrc_ref, wdst_ref) = rest[:14]
    outs = rest[14:]
    if last:
        s_ref, v_ref = outs
        ts_ref = td_ref = None
    else:
        s_ref, v_ref, ts_ref, td_ref = outs
    msum = jnp.concatenate([r[...] for r in strip_refs], axis=-1)
    cnt = jnp.maximum(msum[:, 148:149], 1.0)
    s = sp_ref[...] + msum[:, :100] / cnt
    vf = vp_ref[...] + msum[:, 100:148] / cnt
    s = _ln_scalar(s, g0_ref[0], bb0_ref[0], 100)
    vf = _ln_vec(vf, 16)
    vh = vf @ kf0_ref[...]
    vn = _vnorm3(vh, 32)
    fs = jnp.maximum(jnp.concatenate([s, vn], axis=-1) @ wf0_ref[...] + bf0_ref[0], 0.0)
    fv = vh @ kvf0_ref[...]
    gate = jax.nn.sigmoid(_vnorm3(fv, 32))
    fv = fv * _tile3(gate, 32)
    vh1 = fv @ kf1_ref[...]
    vn1 = _vnorm3(vh1, 32)
    fs1 = jnp.concatenate([fs, vn1], axis=-1) @ wf1_ref[...] + bf1_ref[0]
    fv1 = vh1 @ kvf1_ref[...]
    s = s + fs1
    vf = vf + fv1
    s = _ln_scalar(s, g1_ref[0], bb1_ref[0], 100)
    vf = _ln_vec(vf, 16)
    s_ref[...] = s
    v_ref[...] = vf
    if not last:
        vp = _pack_pairs(vf)
        pad = jnp.zeros((s.shape[0], TW - 124), s.dtype)
        ts_ref[...] = jnp.concatenate([s @ wsrc_ref[...], vp, pad], axis=-1)
        td_ref[...] = jnp.concatenate([s @ wdst_ref[...], vp, pad], axis=-1)


def _out_body(s_ref, v_ref, g_ref, b_ref, wh_ref, wsw_ref, wsb_ref, o_ref):
    s = _ln_scalar(s_ref[...], g_ref[0], b_ref[0], 100)
    vf = _ln_vec(v_ref[...], 16)
    vh = vf @ wh_ref[...]
    vn = _vnorm3(vh, 16)
    so = jnp.concatenate([s, vn], axis=-1) @ wsw_ref[...] + wsb_ref[0]
    o_ref[...] = jnp.maximum(so, 0.0)


def _full(shape):
    return pl.BlockSpec(shape, lambda i: (0,) * len(shape))


def _rows(w, blk=None):
    return pl.BlockSpec((blk or BN, w), lambda i: (i, 0))


def _tc(body, grid, in_specs, out_specs, out_shape):
    return pl.pallas_call(
        body, grid=(grid,), in_specs=in_specs, out_specs=out_specs,
        out_shape=out_shape,
        compiler_params=pltpu.CompilerParams(
            dimension_semantics=("arbitrary",)),
    )


def _sc_gather(tsrc, tdst, srcp, dstp):
    mesh = plsc.VectorSubcoreMesh(core_axis_name="c", subcore_axis_name="s")

    @functools.partial(
        pl.kernel, mesh=mesh,
        out_type=[jax.ShapeDtypeStruct((EPAD, TW), jnp.float32),
                  jax.ShapeDtypeStruct((EPAD, TW), jnp.float32)],
        scratch_types=[pltpu.VMEM((CCH,), jnp.int32),
                       pltpu.VMEM((CCH,), jnp.int32),
                       pltpu.VMEM((CCH, TW), jnp.float32),
                       pltpu.VMEM((CCH, TW), jnp.float32),
                       pltpu.SemaphoreType.DMA,
                       pltpu.SemaphoreType.DMA],
    )
    def k(ts_hbm, td_hbm, src_hbm, dst_hbm, os_hbm, od_hbm,
          idx_s, idx_d, row_s, row_d, sem_s, sem_d):
        wid = lax.axis_index("s") * 2 + lax.axis_index("c")
        base = wid * EPW

        def body(j, carry):
            off = base + j * CCH
            pltpu.sync_copy(src_hbm.at[pl.ds(off, CCH)], idx_s)
            pltpu.sync_copy(dst_hbm.at[pl.ds(off, CCH)], idx_d)
            cp_s = pltpu.async_copy(ts_hbm.at[idx_s], row_s, sem_s)
            cp_d = pltpu.async_copy(td_hbm.at[idx_d], row_d, sem_d)
            cp_s.wait()
            cp_d.wait()
            pltpu.sync_copy(row_s, os_hbm.at[pl.ds(off, CCH)])
            pltpu.sync_copy(row_d, od_hbm.at[pl.ds(off, CCH)])
            return carry

        lax.fori_loop(0, EPW // CCH, body, 0)

    return k(tsrc, tdst, srcp, dstp)


_NBUF = 2
_ZCH = 104
_FCH = 136


def _sc_scatter(dstp, strips, zeros_hbm):
    mesh = plsc.VectorSubcoreMesh(core_axis_name="c", subcore_axis_name="s")
    chunks_per_tile = EPAD // 16 // CCH

    @functools.partial(
        pl.kernel, mesh=mesh,
        out_type=[jax.ShapeDtypeStruct((NACC, SW), jnp.float32)
                  for _ in range(NSTRIP)],
        scratch_types=(
            [pltpu.VMEM((CCH,), jnp.int32) for _ in range(_NBUF)]
            + [pltpu.VMEM((CCH, SW), jnp.float32) for _ in range(_NBUF)]
            + [pltpu.VMEM((_ZCH, SW), jnp.float32),
               pltpu.VMEM((_FCH, SW), jnp.float32),
               pltpu.VMEM_SHARED((ACCR, SW), jnp.float32)]
            + [pltpu.SemaphoreType.DMA for _ in range(2 * _NBUF + 1)]),
    )
    def k(*refs):
        dst_hbm = refs[0]
        ins = refs[1:1 + NSTRIP]
        z_hbm = refs[1 + NSTRIP]
        outs = refs[2 + NSTRIP:2 + 2 * NSTRIP]
        scr = refs[2 + 2 * NSTRIP:]
        idxs = scr[:_NBUF]
        msgs = scr[_NBUF:2 * _NBUF]
        zbuf, fbuf, acc = scr[2 * _NBUF:2 * _NBUF + 3]
        isems = scr[2 * _NBUF + 3:3 * _NBUF + 3]
        msems = scr[3 * _NBUF + 3:4 * _NBUF + 3]
        ssem = scr[4 * _NBUF + 3]
        cid = lax.axis_index("c")
        sid = lax.axis_index("s")
        pltpu.sync_copy(z_hbm, zbuf)
        for st in range(NSTRIP):
            for rr in range(2):
                base = (cid * 2 + rr) * RNG
                tb = sid * (ACCR // 16)
                for h in range(7):
                    pltpu.sync_copy(zbuf, acc.at[pl.ds(tb + h * _ZCH, _ZCH)])
                pltpu.sync_copy(zbuf.at[pl.ds(0, 96)],
                                acc.at[pl.ds(tb + 7 * _ZCH, 96)])
                plsc.subcore_barrier()

                def body(jj, carry, st=st, base=base):
                    cbase = sid * (EPAD // 16) + jj * (_NBUF * CCH)
                    loads = []
                    for b in range(_NBUF):
                        off = cbase + b * CCH
                        li = pltpu.async_copy(dst_hbm.at[pl.ds(off, CCH)],
                                              idxs[b], isems[b])
                        lm = pltpu.async_copy(ins[st].at[pl.ds(off, CCH)],
                                              msgs[b], msems[b])
                        loads.append((li, lm))
                    adds = []
                    for b in range(_NBUF):
                        li, lm = loads[b]
                        li.wait()
                        for q in range(CCH // 16):
                            v = idxs[b][pl.ds(q * 16, 16)]
                            rel = v - base
                            oob = (rel < 0) | (rel >= RNG)
                            idxs[b][pl.ds(q * 16, 16)] = jnp.where(
                                oob, RNG + (v & 127), rel)
                        lm.wait()
                        adds.append(pltpu.async_copy(msgs[b], acc.at[idxs[b]],
                                                     ssem, add=True))
                    for b in range(_NBUF):
                        adds[b].wait()
                    return carry

                lax.fori_loop(0, chunks_per_tile // _NBUF, body, 0)
                plsc.subcore_barrier()
                for h in range(6):
                    arow = sid * (6 * _FCH) + h * _FCH
                    pltpu.sync_copy(acc.at[pl.ds(arow, _FCH)], fbuf)
                    pltpu.sync_copy(fbuf, outs[st].at[pl.ds(base + arow, _FCH)])
                plsc.subcore_barrier()

    return k(dstp, *strips, zeros_hbm)


def _sc_scatter_bisect(dstp, strips, zeros_hbm):
    del zeros_hbm
    return [jax.ops.segment_sum(st, dstp, num_segments=NACC) for st in strips]


def _pad_rows(x, n):
    return jnp.pad(x, ((0, n - x.shape[0]),) + ((0, 0),) * (x.ndim - 1))


def kernel(params, h_V_s, h_V_v, edge_index, h_E_s, h_E_v, seq):
    f32 = jnp.float32
    p = params
    ngrid = NPAD // BN
    egrid = EPAD // BE

    hs = _pad_rows(h_V_s, NPAD)
    hs = jnp.pad(hs, ((0, 0), (0, 2)))
    vf0 = _pad_rows(h_V_v.transpose(0, 2, 1).reshape(-1, 9), NPAD)
    vf0 = jnp.pad(vf0, ((0, 0), (0, 7)))
    seqp = _pad_rows(seq[:, None], NPAD)[:, 0]
    soh = (seqp[:, None] == jnp.arange(24, dtype=jnp.int32)[None, :]).astype(f32)
    es_in = _pad_rows(h_E_s, EPAD)
    evf = _pad_rows(h_E_v[:, 0, :], EPAD)
    evf = jnp.pad(evf, ((0, 0), (0, 5)))
    srcp = jnp.pad(edge_index[0], (0, EPAD - N_EDGES_K))
    npad_e = EPAD - N_EDGES_K
    pad_rows = N_NODES_K + jnp.arange(npad_e, dtype=jnp.int32) % (NACC - N_NODES_K)
    dstp = jnp.concatenate([edge_index[1], pad_rows])
    zeros_hbm = jnp.zeros((_ZCH, SW), f32)

    wsemb = jnp.pad(p["W_s"], ((0, 4), (0, 0)))

    gvp0 = p["Wv_gvp"]
    node0_w = (wsemb,
               p["Wv_ln"]["g"][None, :].astype(f32), p["Wv_ln"]["b"][None, :],
               _kron3(gvp0["wh"]["w"]),
               gvp0["ws"]["w"].T, gvp0["ws"]["b"][None, :],
               _kron3(gvp0["wv"]["w"]))

    gvpe = p["We_gvp"]
    edge0_w = (p["We_ln"]["g"][None, :], p["We_ln"]["b"][None, :],
               gvpe["wh"]["w"], gvpe["ws"]["w"].T, gvpe["ws"]["b"][None, :],
               gvpe["wv"]["w"])

    def layer_w(i):
        lp = p["layer%d" % i]["conv"]
        m0, m1, m2 = lp["m0"], lp["m1"], lp["m2"]
        wh0 = m0["wh"]["w"]
        ws0 = m0["ws"]["w"]
        return dict(
            wsrc=ws0[:, 0:100].T, wdst=ws0[:, 132:232].T,
            wes=ws0[:, 100:132].T,
            wd0=ws0[:, 232:265].T,
            k0s=_kron3(wh0[:, 0:16]),
            k0e=jnp.kron(jnp.eye(3, dtype=f32), wh0[:, 16:17].T),
            k0d=_kron3(wh0[:, 17:33]),
            b0v=m0["ws"]["b"][None, :],
            kv0=_kron3(m0["wv"]["w"]),
            k1h=_kron3(m1["wh"]["w"]),
            w1s=m1["ws"]["w"][:, 0:100].T, w1n=m1["ws"]["w"][:, 100:116].T,
            b1=m1["ws"]["b"][None, :],
            kv1=_kron3(m1["wv"]["w"]),
            k2h=_kron3(m2["wh"]["w"]),
            w2s=m2["ws"]["w"][:, 0:100].T, w2n=m2["ws"]["w"][:, 100:116].T,
            b2=m2["ws"]["b"][None, :],
            kv2=_kron3(m2["wv"]["w"]),
        )

    def ff_w(i, nxt):
        lp = p["layer%d" % i]
        f0, f1 = lp["ff0"], lp["ff1"]
        w = dict(
            g0=lp["norm0"]["g"][None, :], bb0=lp["norm0"]["b"][None, :],
            kf0=_kron3(f0["wh"]["w"]),
            wf0=f0["ws"]["w"].T, bf0=f0["ws"]["b"][None, :],
            kvf0=_kron3(f0["wv"]["w"]),
            kf1=_kron3(f1["wh"]["w"]),
            wf1=f1["ws"]["w"].T, bf1=f1["ws"]["b"][None, :],
            kvf1=_kron3(f1["wv"]["w"]),
            g1=lp["norm1"]["g"][None, :], bb1=lp["norm1"]["b"][None, :],
        )
        if nxt is not None:
            ws0n = p["layer%d" % nxt]["conv"]["m0"]["ws"]["w"]
            w["wsrc"] = ws0n[:, 0:100].T
            w["wdst"] = ws0n[:, 132:232].T
        else:
            w["wsrc"] = jnp.zeros((100, 100), f32)
            w["wdst"] = jnp.zeros((100, 100), f32)
        return w

    lw0 = layer_w(0)
    s, v, tsrc, tdst = _tc(
        _node0_body, ngrid,
        [_rows(8), _rows(24), _rows(16)] + [_full(w.shape) for w in (
            node0_w[0], node0_w[1], node0_w[2], node0_w[3], node0_w[4],
            node0_w[5], node0_w[6], lw0["wsrc"], lw0["wdst"])],
        [_rows(100), _rows(48), _rows(TW), _rows(TW)],
        [jax.ShapeDtypeStruct((NPAD, 100), f32),
         jax.ShapeDtypeStruct((NPAD, 48), f32),
         jax.ShapeDtypeStruct((NPAD, TW), f32),
         jax.ShapeDtypeStruct((NPAD, TW), f32)],
    )(hs, soh, vf0, *node0_w, lw0["wsrc"], lw0["wdst"])

    ef = _tc(
        _edge0_body, egrid,
        [_rows(32, BE), _rows(8, BE)] + [_full(w.shape) for w in edge0_w],
        [_rows(64, BE)],
        [jax.ShapeDtypeStruct((EPAD, 64), f32)],
    )(es_in, evf, *edge0_w)[0]

    for i in range(3):
        lw = layer_w(i)
        gs, gd = _sc_gather(tsrc, tdst, srcp, dstp)
        mw = [lw[k] for k in ("k0s", "k0e", "k0d", "wes", "wd0", "b0v", "kv0",
                              "k1h", "w1s", "w1n", "b1", "kv1",
                              "k2h", "w2s", "w2n", "b2", "kv2")]
        strips = _tc(
            _msg_body, egrid,
            [_rows(TW, BE), _rows(TW, BE), _rows(64, BE)] +
            [_full(w.shape) for w in mw],
            [_rows(SW, BE)] * NSTRIP,
            [jax.ShapeDtypeStruct((EPAD, SW), f32)] * NSTRIP,
        )(gs, gd, ef, *mw)
        acc = _sc_scatter_bisect(dstp, strips, zeros_hbm)
        accs = [a[:NPAD] for a in acc]
        fw = ff_w(i, i + 1 if i < 2 else None)
        uw = [fw[k] for k in ("g0", "bb0", "kf0", "wf0", "bf0", "kvf0",
                              "kf1", "wf1", "bf1", "kvf1", "g1", "bb1",
                              "wsrc", "wdst")]
        last = i == 2
        outs = _tc(
            functools.partial(_upd_body, last), ngrid,
            [_rows(100), _rows(48)] + [_rows(SW)] * NSTRIP +
            [_full(w.shape) for w in uw],
            [_rows(100), _rows(48)] + ([] if last else [_rows(TW), _rows(TW)]),
            [jax.ShapeDtypeStruct((NPAD, 100), f32),
             jax.ShapeDtypeStruct((NPAD, 48), f32)] +
            ([] if last else [jax.ShapeDtypeStruct((NPAD, TW), f32),
                              jax.ShapeDtypeStruct((NPAD, TW), f32)]),
        )(s, v, *accs, *uw)
        if last:
            s, v = outs
        else:
            s, v, tsrc, tdst = outs

    og = p["Wout_gvp"]
    ow = (p["Wout_ln"]["g"][None, :], p["Wout_ln"]["b"][None, :],
          _kron3(og["wh"]["w"]), og["ws"]["w"].T, og["ws"]["b"][None, :])
    out = _tc(
        _out_body, ngrid,
        [_rows(100), _rows(48)] + [_full(w.shape) for w in ow],
        [_rows(100)],
        [jax.ShapeDtypeStruct((NPAD, 100), f32)],
    )(s, v, *ow)[0]
    return out[:N_NODES_K]
